# Initial kernel scaffold; baseline (speedup 1.0000x reference)
import jax
import jax.numpy as jnp
from jax.experimental import pallas as pl


def kernel(x, edge_index, W1, b1, W2, b2):
    raise NotImplementedError("write your pallas kernel here")


# speedup vs baseline: 7.1694x; 7.1694x over previous
import functools

import jax
import jax.numpy as jnp
from jax import lax
from jax.experimental import pallas as pl
from jax.experimental.pallas import tpu as pltpu
from jax.experimental.pallas import tpu_sc as plsc

NC = 2
NS = 16
DW = 128


def _agg_call(n, na, e, dh, kc, zr):
  nch = e // (NS * kc)
  ept = e // NS
  rpt = na // NS
  nz = rpt // zr
  mesh = plsc.VectorSubcoreMesh(core_axis_name="c", subcore_axis_name="s",
                                num_cores=NC, num_subcores=NS)

  @functools.partial(
      pl.kernel,
      out_type=jax.ShapeDtypeStruct((NC * na, dh), jnp.float32),
      mesh=mesh,
      scratch_types=[
          pltpu.VMEM((kc,), jnp.int32),
          pltpu.VMEM((kc,), jnp.int32),
          pltpu.VMEM((kc, dh), jnp.float32),
          pltpu.VMEM((zr, dh), jnp.float32),
          pltpu.VMEM_SHARED((na, dh), jnp.float32),
          pltpu.SemaphoreType.DMA,
      ],
  )
  def agg(p2, src2, dst_hbm, zeros_hbm, out2,
          src_v, dst_v, buf, zbuf, acc, sem):
    c = lax.axis_index("c")
    s = lax.axis_index("s")
    pltpu.sync_copy(zeros_hbm, zbuf)
    for k in range(nz):
      pltpu.sync_copy(zbuf, acc.at[pl.ds(s * rpt + k * zr, zr)])
    plsc.subcore_barrier()

    sbase = c * e + s * ept
    dbase = s * ept

    def body(i, _):
      pltpu.sync_copy(src2.at[pl.ds(sbase + i * kc, kc)], src_v)
      pltpu.sync_copy(dst_hbm.at[pl.ds(dbase + i * kc, kc)], dst_v)
      pltpu.async_copy(p2.at[src_v], buf, sem).wait()
      pltpu.sync_copy(buf, acc.at[dst_v], add=True)
      return 0
    lax.fori_loop(0, nch, body, 0)
    plsc.subcore_barrier()

    for k in range(nz):
      r0 = s * rpt + k * zr
      pltpu.sync_copy(acc.at[pl.ds(r0, zr)], zbuf)
      pltpu.sync_copy(zbuf, out2.at[pl.ds(c * na + r0, zr)])

  return agg


def _deg_call(n, na, e, kc, zr):
  eh = e // NC
  ept = eh // NS
  nch = eh // (NS * kc)
  rpt = na // NS
  nz = rpt // zr
  mesh = plsc.VectorSubcoreMesh(core_axis_name="c", subcore_axis_name="s",
                                num_cores=NC, num_subcores=NS)

  @functools.partial(
      pl.kernel,
      out_type=jax.ShapeDtypeStruct((NC * na, DW), jnp.float32),
      mesh=mesh,
      scratch_types=[
          pltpu.VMEM((kc,), jnp.int32),
          pltpu.VMEM((kc, DW), jnp.float32),
          pltpu.VMEM((zr, DW), jnp.float32),
          pltpu.VMEM_SHARED((na, DW), jnp.float32),
      ],
  )
  def deg(dst_hbm, ones_hbm, zeros_hbm, degp, dst_v, ones_v, dbuf, acc):
    c = lax.axis_index("c")
    s = lax.axis_index("s")
    pltpu.sync_copy(ones_hbm, ones_v)
    pltpu.sync_copy(zeros_hbm, dbuf)
    for k in range(nz):
      pltpu.sync_copy(dbuf, acc.at[pl.ds(s * rpt + k * zr, zr)])
    plsc.subcore_barrier()

    base = c * eh + s * ept

    def body(i, _):
      pltpu.sync_copy(dst_hbm.at[pl.ds(base + i * kc, kc)], dst_v)
      pltpu.sync_copy(ones_v, acc.at[dst_v], add=True)
      return 0
    lax.fori_loop(0, nch, body, 0)
    plsc.subcore_barrier()

    for k in range(nz):
      r0 = s * rpt + k * zr
      pltpu.sync_copy(acc.at[pl.ds(r0, zr)], dbuf)
      pltpu.sync_copy(dbuf, degp.at[pl.ds(c * na + r0, zr)])

  return deg


def _dinv_block(d0, d1):
  deg = d0[0, :, 0:1] + d1[0, :, 0:1] + 1.0
  return lax.rsqrt(deg)


def _mm1_body(x_ref, w_ref, d0_ref, d1_ref, p_ref):
  dinv = _dinv_block(d0_ref[...], d1_ref[...])
  h = jnp.dot(x_ref[...], w_ref[...], preferred_element_type=jnp.float32)
  p = h * dinv
  p_ref[0] = p[:, :128]
  p_ref[1] = p[:, 128:]


def _mm2_body(a_ref, p_ref, d0_ref, d1_ref, b_ref, w_ref, q_ref):
  dinv = _dinv_block(d0_ref[...], d1_ref[...])
  b = b_ref[...]
  z_lo = jnp.maximum(dinv * (a_ref[0] + p_ref[0]) + b[:, :128], 0.0)
  z_hi = jnp.maximum(dinv * (a_ref[1] + p_ref[1]) + b[:, 128:], 0.0)
  z = jnp.concatenate([z_lo, z_hi], axis=1)
  h = jnp.dot(z, w_ref[...], preferred_element_type=jnp.float32)
  q = h * dinv
  q_ref[0] = q[:, :128]
  q_ref[1] = q[:, 128:]


def _fin_body(a_ref, q_ref, d0_ref, d1_ref, b_ref, out_ref):
  dinv = _dinv_block(d0_ref[...], d1_ref[...])
  b = b_ref[...]
  o_lo = dinv * (a_ref[0] + q_ref[0]) + b[:, :128]
  o_hi = dinv * (a_ref[1] + q_ref[1]) + b[:, 128:]
  out_ref[...] = jnp.concatenate([o_lo, o_hi], axis=1)


def kernel(x, edge_index, W1, b1, W2, b2):
  n, d_in = x.shape
  e = edge_index.shape[1]
  dh = 128
  assert d_in == 256 and W1.shape == (256, 256) and W2.shape == (256, 256)
  na = ((n + NS * 8 - 1) // (NS * 8)) * (NS * 8)
  zr = 128
  while (na // NS) % zr:
    zr //= 2
  assert e % (NS * 80) == 0 and (e // NC) % (NS * 40) == 0

  src = edge_index[0].astype(jnp.int32)
  dst = edge_index[1].astype(jnp.int32)
  src2 = jnp.concatenate([src, src + n])
  b1r = b1.reshape(1, 256)
  b2r = b2.reshape(1, 256)

  zeros_w = jnp.zeros((zr, dh), jnp.float32)
  ones_d = jnp.ones((40, DW), jnp.float32)

  degp = _deg_call(n, na, e, 40, zr)(dst, ones_d, zeros_w).reshape(NC, na, DW)

  bm = 1000
  grid = (n // bm,)
  row_spec = pl.BlockSpec((bm, 256), lambda i: (i, 0))
  deg0_spec = pl.BlockSpec((1, bm, DW), lambda i: (0, i, 0))
  deg1_spec = pl.BlockSpec((1, bm, DW), lambda i: (1, i, 0))
  p_spec = pl.BlockSpec((2, bm, dh), lambda i: (0, i, 0))
  w_spec = pl.BlockSpec((256, 256), lambda i: (0, 0))
  b_spec = pl.BlockSpec((1, 256), lambda i: (0, 0))
  half_shape = jax.ShapeDtypeStruct((2, n, dh), jnp.float32)

  p_all = pl.pallas_call(
      _mm1_body,
      grid=grid,
      in_specs=[row_spec, w_spec, deg0_spec, deg1_spec],
      out_specs=p_spec,
      out_shape=half_shape,
  )(x, W1, degp, degp)

  agg = _agg_call(n, na, e, dh, 80, zr)
  a1 = agg(p_all.reshape(2 * n, dh), src2, dst, zeros_w).reshape(NC, na, dh)

  q_all = pl.pallas_call(
      _mm2_body,
      grid=grid,
      in_specs=[p_spec, p_spec, deg0_spec, deg1_spec, b_spec, w_spec],
      out_specs=p_spec,
      out_shape=half_shape,
  )(a1, p_all, degp, degp, b1r, W2)

  a2 = agg(q_all.reshape(2 * n, dh), src2, dst, zeros_w).reshape(NC, na, dh)

  out = pl.pallas_call(
      _fin_body,
      grid=grid,
      in_specs=[p_spec, p_spec, deg0_spec, deg1_spec, b_spec],
      out_specs=row_spec,
      out_shape=jax.ShapeDtypeStruct((n, 256), jnp.float32),
  )(a2, q_all, degp, degp, b2r)

  return out


# speedup vs baseline: 8.0293x; 1.1199x over previous
import functools

import jax
import jax.numpy as jnp
from jax import lax
from jax.experimental import pallas as pl
from jax.experimental.pallas import tpu as pltpu
from jax.experimental.pallas import tpu_sc as plsc

NC = 2
NS = 16
DW = 128
DEPTH = 5


def _agg_call(n, na, e, dh, kc, zr):
  nch = e // (NS * kc)
  ept = e // NS
  assert nch % DEPTH == 0
  rpt = na // NS
  nz = rpt // zr
  mesh = plsc.VectorSubcoreMesh(core_axis_name="c", subcore_axis_name="s",
                                num_cores=NC, num_subcores=NS)

  @functools.partial(
      pl.kernel,
      out_type=jax.ShapeDtypeStruct((NC * na, dh), jnp.float32),
      mesh=mesh,
      scratch_types=[
          pltpu.VMEM((DEPTH, kc), jnp.int32),
          pltpu.VMEM((DEPTH, kc), jnp.int32),
          pltpu.VMEM((DEPTH, kc, dh), jnp.float32),
          pltpu.VMEM((zr, dh), jnp.float32),
          pltpu.VMEM_SHARED((na, dh), jnp.float32),
      ] + [pltpu.SemaphoreType.DMA] * DEPTH,
  )
  def agg(p2, src2, dst_hbm, zeros_hbm, out2,
          src_v, dst_v, buf, zbuf, acc, *sems):
    c = lax.axis_index("c")
    s = lax.axis_index("s")
    pltpu.sync_copy(zeros_hbm, zbuf)
    for k in range(nz):
      pltpu.sync_copy(zbuf, acc.at[pl.ds(s * rpt + k * zr, zr)])
    plsc.subcore_barrier()

    sbase = c * e + s * ept
    dbase = s * ept

    def stage(chunk, b):
      pltpu.sync_copy(src2.at[pl.ds(sbase + chunk * kc, kc)], src_v.at[b])
      pltpu.sync_copy(dst_hbm.at[pl.ds(dbase + chunk * kc, kc)], dst_v.at[b])
      pltpu.async_copy(p2.at[src_v.at[b]], buf.at[b], sems[b])

    def finish(b):
      pltpu.make_async_copy(p2.at[src_v.at[b]], buf.at[b], sems[b]).wait()
      pltpu.sync_copy(buf.at[b], acc.at[dst_v.at[b]], add=True)

    for b in range(DEPTH):
      stage(b, b)

    @pl.loop(0, nch - DEPTH, step=DEPTH)
    def _pipe(g):
      for b in range(DEPTH):
        finish(b)
        stage(g + DEPTH + b, b)

    for b in range(DEPTH):
      finish(b)
    plsc.subcore_barrier()

    for k in range(nz):
      r0 = s * rpt + k * zr
      pltpu.sync_copy(acc.at[pl.ds(r0, zr)], zbuf)
      pltpu.sync_copy(zbuf, out2.at[pl.ds(c * na + r0, zr)])

  return agg


def _deg_call(n, na, e, kc, zr):
  eh = e // NC
  ept = eh // NS
  nch = eh // (NS * kc)
  assert nch % DEPTH == 0
  rpt = na // NS
  nz = rpt // zr
  mesh = plsc.VectorSubcoreMesh(core_axis_name="c", subcore_axis_name="s",
                                num_cores=NC, num_subcores=NS)

  @functools.partial(
      pl.kernel,
      out_type=jax.ShapeDtypeStruct((NC * na, DW), jnp.float32),
      mesh=mesh,
      scratch_types=[
          pltpu.VMEM((DEPTH, kc), jnp.int32),
          pltpu.VMEM((kc, DW), jnp.float32),
          pltpu.VMEM((zr, DW), jnp.float32),
          pltpu.VMEM_SHARED((na, DW), jnp.float32),
      ] + [pltpu.SemaphoreType.DMA] * DEPTH,
  )
  def deg(dst_hbm, ones_hbm, zeros_hbm, degp, dst_v, ones_v, dbuf, acc, *sems):
    c = lax.axis_index("c")
    s = lax.axis_index("s")
    pltpu.sync_copy(ones_hbm, ones_v)
    pltpu.sync_copy(zeros_hbm, dbuf)
    for k in range(nz):
      pltpu.sync_copy(dbuf, acc.at[pl.ds(s * rpt + k * zr, zr)])
    plsc.subcore_barrier()

    base = c * eh + s * ept

    def stage(chunk, b):
      pltpu.async_copy(dst_hbm.at[pl.ds(base + chunk * kc, kc)],
                       dst_v.at[b], sems[b])

    def finish(b):
      pltpu.make_async_copy(dst_hbm.at[pl.ds(base, kc)],
                            dst_v.at[b], sems[b]).wait()
      pltpu.sync_copy(ones_v, acc.at[dst_v.at[b]], add=True)

    for b in range(DEPTH):
      stage(b, b)

    @pl.loop(0, nch - DEPTH, step=DEPTH)
    def _pipe(g):
      for b in range(DEPTH):
        finish(b)
        stage(g + DEPTH + b, b)

    for b in range(DEPTH):
      finish(b)
    plsc.subcore_barrier()

    for k in range(nz):
      r0 = s * rpt + k * zr
      pltpu.sync_copy(acc.at[pl.ds(r0, zr)], dbuf)
      pltpu.sync_copy(dbuf, degp.at[pl.ds(c * na + r0, zr)])

  return deg


def _dinv_block(d0, d1):
  deg = d0[0, :, 0:1] + d1[0, :, 0:1] + 1.0
  return lax.rsqrt(deg)


def _mm1_body(x_ref, w_ref, d0_ref, d1_ref, p_ref):
  dinv = _dinv_block(d0_ref[...], d1_ref[...])
  h = jnp.dot(x_ref[...], w_ref[...], preferred_element_type=jnp.float32)
  p = h * dinv
  p_ref[0] = p[:, :128]
  p_ref[1] = p[:, 128:]


def _mm2_body(a_ref, p_ref, d0_ref, d1_ref, b_ref, w_ref, q_ref):
  dinv = _dinv_block(d0_ref[...], d1_ref[...])
  b = b_ref[...]
  z_lo = jnp.maximum(dinv * (a_ref[0] + p_ref[0]) + b[:, :128], 0.0)
  z_hi = jnp.maximum(dinv * (a_ref[1] + p_ref[1]) + b[:, 128:], 0.0)
  z = jnp.concatenate([z_lo, z_hi], axis=1)
  h = jnp.dot(z, w_ref[...], preferred_element_type=jnp.float32)
  q = h * dinv
  q_ref[0] = q[:, :128]
  q_ref[1] = q[:, 128:]


def _fin_body(a_ref, q_ref, d0_ref, d1_ref, b_ref, out_ref):
  dinv = _dinv_block(d0_ref[...], d1_ref[...])
  b = b_ref[...]
  o_lo = dinv * (a_ref[0] + q_ref[0]) + b[:, :128]
  o_hi = dinv * (a_ref[1] + q_ref[1]) + b[:, 128:]
  out_ref[...] = jnp.concatenate([o_lo, o_hi], axis=1)


def kernel(x, edge_index, W1, b1, W2, b2):
  n, d_in = x.shape
  e = edge_index.shape[1]
  dh = 128
  assert d_in == 256 and W1.shape == (256, 256) and W2.shape == (256, 256)
  na = ((n + NS * 8 - 1) // (NS * 8)) * (NS * 8)
  zr = 64
  while (na // NS) % zr:
    zr //= 2
  assert e % (NS * 40 * DEPTH) == 0 and (e // NC) % (NS * 40 * DEPTH) == 0

  src = edge_index[0].astype(jnp.int32)
  dst = edge_index[1].astype(jnp.int32)
  src2 = jnp.concatenate([src, src + n])
  b1r = b1.reshape(1, 256)
  b2r = b2.reshape(1, 256)

  zeros_w = jnp.zeros((zr, dh), jnp.float32)
  ones_d = jnp.ones((40, DW), jnp.float32)

  degp = _deg_call(n, na, e, 40, zr)(dst, ones_d, zeros_w).reshape(NC, na, DW)

  bm = 1000
  grid = (n // bm,)
  row_spec = pl.BlockSpec((bm, 256), lambda i: (i, 0))
  deg0_spec = pl.BlockSpec((1, bm, DW), lambda i: (0, i, 0))
  deg1_spec = pl.BlockSpec((1, bm, DW), lambda i: (1, i, 0))
  p_spec = pl.BlockSpec((2, bm, dh), lambda i: (0, i, 0))
  w_spec = pl.BlockSpec((256, 256), lambda i: (0, 0))
  b_spec = pl.BlockSpec((1, 256), lambda i: (0, 0))
  half_shape = jax.ShapeDtypeStruct((2, n, dh), jnp.float32)

  p_all = pl.pallas_call(
      _mm1_body,
      grid=grid,
      in_specs=[row_spec, w_spec, deg0_spec, deg1_spec],
      out_specs=p_spec,
      out_shape=half_shape,
  )(x, W1, degp, degp)

  agg = _agg_call(n, na, e, dh, 40, zr)
  a1 = agg(p_all.reshape(2 * n, dh), src2, dst, zeros_w).reshape(NC, na, dh)

  q_all = pl.pallas_call(
      _mm2_body,
      grid=grid,
      in_specs=[p_spec, p_spec, deg0_spec, deg1_spec, b_spec, w_spec],
      out_specs=p_spec,
      out_shape=half_shape,
  )(a1, p_all, degp, degp, b1r, W2)

  a2 = agg(q_all.reshape(2 * n, dh), src2, dst, zeros_w).reshape(NC, na, dh)

  out = pl.pallas_call(
      _fin_body,
      grid=grid,
      in_specs=[p_spec, p_spec, deg0_spec, deg1_spec, b_spec],
      out_specs=row_spec,
      out_shape=jax.ShapeDtypeStruct((n, 256), jnp.float32),
  )(a2, q_all, degp, degp, b2r)

  return out
